# baseline (device time: 22128 ns/iter reference)
import jax
import jax.numpy as jnp
from jax import lax
from jax.experimental import pallas as pl
from jax.experimental.pallas import tpu as pltpu

K = 1024
H = 512
D = 1024
SIZES = (32, 64, 96, 96, 96, 64, 32, 32)
OFFS = tuple(sum(SIZES[:i]) for i in range(len(SIZES)))
C = len(SIZES)
assert sum(SIZES) == H


def kernel(partial, gamma):
    g = gamma.reshape(1, D)
    partial = pltpu.with_memory_space_constraint(
        partial, pltpu.MemorySpace.HBM)
    g = pltpu.with_memory_space_constraint(g, pltpu.MemorySpace.HBM)

    def body(p_ref, g_ref, out_ref, send_src, send_buf, recv_direct,
             recv_fwd, local_buf, out_vmem, g_vmem,
             load_sems, local_sems, out_sems, g_sem,
             y_send_sems, y_recv_sems, x_send_sems, x_recv_sems):
        my_x = lax.axis_index("x")
        my_y = lax.axis_index("y")
        other_x = 1 - my_x
        other_y = 1 - my_y

        send_row0 = other_y * K + my_x * H
        base = my_y * K
        off_d = my_x * H
        off_f = other_x * H

        g_cp = pltpu.make_async_copy(g_ref, g_vmem, g_sem)
        g_cp.start()
        loads = []
        for i in range(C):
            cp = pltpu.make_async_copy(
                p_ref.at[0, pl.ds(send_row0 + OFFS[i], SIZES[i]), :],
                send_src.at[pl.ds(OFFS[i], SIZES[i]), :],
                load_sems.at[i])
            cp.start()
            loads.append(cp)
        local_cp = []
        for j, off in enumerate((off_d, off_f)):
            cp = pltpu.make_async_copy(
                p_ref.at[0, pl.ds(base + off, H), :],
                local_buf.at[pl.ds(off, H), :],
                local_sems.at[j])
            cp.start()
            local_cp.append(cp)

        barrier = pltpu.get_barrier_semaphore()
        pl.semaphore_signal(barrier, inc=1, device_id=(my_x, other_y),
                            device_id_type=pl.DeviceIdType.MESH)
        pl.semaphore_signal(barrier, inc=1, device_id=(other_x, my_y),
                            device_id_type=pl.DeviceIdType.MESH)
        pl.semaphore_wait(barrier, 2)

        out_cps = []

        def fold(recv_ref, i, off, sem_i):
            r = pl.ds(OFFS[i], SIZES[i])
            ro = pl.ds(off + OFFS[i], SIZES[i])
            yc = (local_buf[ro, :] + recv_ref[r, :].astype(jnp.float32))
            inv = lax.rsqrt(jnp.mean(yc * yc, axis=-1, keepdims=True) + 1e-6)
            out_vmem[ro, :] = (yc * inv * g_vmem[...]).astype(jnp.bfloat16)
            cp = pltpu.make_async_copy(
                out_vmem.at[ro, :], out_ref.at[ro, :], out_sems.at[sem_i])
            cp.start()
            out_cps.append(cp)

        rdma_y = []
        for i in range(C):
            r = pl.ds(OFFS[i], SIZES[i])
            loads[i].wait()
            send_buf[r, :] = send_src[r, :].astype(jnp.bfloat16)
            rdma = pltpu.make_async_remote_copy(
                src_ref=send_buf.at[r], dst_ref=recv_direct.at[r],
                send_sem=y_send_sems.at[i], recv_sem=y_recv_sems.at[i],
                device_id=(my_x, other_y),
                device_id_type=pl.DeviceIdType.MESH)
            rdma.start()
            rdma_y.append(rdma)

        g_cp.wait()
        local_cp[0].wait()

        rdma_x = []
        for i in range(C):
            r = pl.ds(OFFS[i], SIZES[i])
            rdma_y[i].wait_recv()
            rdma = pltpu.make_async_remote_copy(
                src_ref=recv_direct.at[r], dst_ref=recv_fwd.at[r],
                send_sem=x_send_sems.at[i], recv_sem=x_recv_sems.at[i],
                device_id=(other_x, my_y),
                device_id_type=pl.DeviceIdType.MESH)
            rdma.start()
            rdma_x.append(rdma)
            fold(recv_direct, i, off_d, i)

        local_cp[1].wait()

        for i in range(C):
            rdma_x[i].wait_recv()
            fold(recv_fwd, i, off_f, C + i)

        for cp in out_cps:
            cp.wait()
        for i in range(C):
            rdma_y[i].wait_send()
            rdma_x[i].wait_send()

    return pl.pallas_call(
        body,
        out_shape=jax.ShapeDtypeStruct((K, D), jnp.bfloat16),
        in_specs=[pl.BlockSpec(memory_space=pltpu.MemorySpace.HBM),
                  pl.BlockSpec(memory_space=pltpu.MemorySpace.HBM)],
        out_specs=pl.BlockSpec(memory_space=pltpu.MemorySpace.HBM),
        scratch_shapes=[
            pltpu.VMEM((H, D), jnp.float32),
            pltpu.VMEM((H, D), jnp.bfloat16),
            pltpu.VMEM((H, D), jnp.bfloat16),
            pltpu.VMEM((H, D), jnp.bfloat16),
            pltpu.VMEM((K, D), jnp.float32),
            pltpu.VMEM((K, D), jnp.bfloat16),
            pltpu.VMEM((1, D), jnp.float32),
            pltpu.SemaphoreType.DMA((C,)),
            pltpu.SemaphoreType.DMA((2,)),
            pltpu.SemaphoreType.DMA((2 * C,)),
            pltpu.SemaphoreType.DMA,
            pltpu.SemaphoreType.DMA((C,)),
            pltpu.SemaphoreType.DMA((C,)),
            pltpu.SemaphoreType.DMA((C,)),
            pltpu.SemaphoreType.DMA((C,)),
        ],
        compiler_params=pltpu.CompilerParams(collective_id=0),
    )(partial, g)


# device time: 21599 ns/iter; 1.0245x vs baseline; 1.0245x over previous
import jax
import jax.numpy as jnp
from jax import lax
from jax.experimental import pallas as pl
from jax.experimental.pallas import tpu as pltpu

K = 1024
D = 1024
CH = 64
NA = 9
NB = 7
SPLIT = NA * CH


def kernel(partial, gamma):
    g = gamma.reshape(1, D)
    partial = pltpu.with_memory_space_constraint(
        partial, pltpu.MemorySpace.HBM)
    g = pltpu.with_memory_space_constraint(g, pltpu.MemorySpace.HBM)

    def body(p_ref, g_ref, out_ref, send_src, send_buf, recv_direct,
             recv_fwd, local_buf, out_vmem, g_vmem,
             load_sems, local_sems, out_sems, g_sem,
             y_send_sems, y_recv_sems, x_send_sems, x_recv_sems):
        my_x = lax.axis_index("x")
        my_y = lax.axis_index("y")
        other_x = 1 - my_x
        other_y = 1 - my_y
        is_a = my_x == 0

        send_row0 = other_y * K + my_x * SPLIT
        base = my_y * K
        off_d = my_x * SPLIT
        off_f = other_x * SPLIT

        def load(i):
            return pltpu.make_async_copy(
                p_ref.at[0, pl.ds(send_row0 + i * CH, CH), :],
                send_src.at[pl.ds(i * CH, CH), :],
                load_sems.at[i])

        def y_rdma(i):
            r = pl.ds(i * CH, CH)
            return pltpu.make_async_remote_copy(
                src_ref=send_buf.at[r], dst_ref=recv_direct.at[r],
                send_sem=y_send_sems.at[i], recv_sem=y_recv_sems.at[i],
                device_id=(my_x, other_y),
                device_id_type=pl.DeviceIdType.MESH)

        def x_rdma(i):
            r = pl.ds(i * CH, CH)
            return pltpu.make_async_remote_copy(
                src_ref=recv_direct.at[r], dst_ref=recv_fwd.at[r],
                send_sem=x_send_sems.at[i], recv_sem=x_recv_sems.at[i],
                device_id=(other_x, my_y),
                device_id_type=pl.DeviceIdType.MESH)

        def out_cp(i, off, sem_i):
            ro = pl.ds(off + i * CH, CH)
            return pltpu.make_async_copy(
                out_vmem.at[ro, :], out_ref.at[ro, :], out_sems.at[sem_i])

        g_cp = pltpu.make_async_copy(g_ref, g_vmem, g_sem)
        g_cp.start()
        for i in range(NB):
            load(i).start()

        @pl.when(is_a)
        def _():
            for i in range(NB, NA):
                load(i).start()

        la = pltpu.make_async_copy(
            p_ref.at[0, pl.ds(base, SPLIT), :],
            local_buf.at[pl.ds(0, SPLIT), :], local_sems.at[0])
        lb = pltpu.make_async_copy(
            p_ref.at[0, pl.ds(base + SPLIT, K - SPLIT), :],
            local_buf.at[pl.ds(SPLIT, K - SPLIT), :], local_sems.at[1])
        la.start()
        lb.start()

        barrier = pltpu.get_barrier_semaphore()
        pl.semaphore_signal(barrier, inc=1, device_id=(my_x, other_y),
                            device_id_type=pl.DeviceIdType.MESH)
        pl.semaphore_signal(barrier, inc=1, device_id=(other_x, my_y),
                            device_id_type=pl.DeviceIdType.MESH)
        pl.semaphore_wait(barrier, 2)

        def send_chunk(i):
            r = pl.ds(i * CH, CH)
            load(i).wait()
            send_buf[r, :] = send_src[r, :].astype(jnp.bfloat16)
            y_rdma(i).start()

        for i in range(NB):
            send_chunk(i)

        @pl.when(is_a)
        def _():
            for i in range(NB, NA):
                send_chunk(i)

        g_cp.wait()
        la.wait()
        lb.wait()

        def fold(recv_ref, i, off, sem_i):
            r = pl.ds(i * CH, CH)
            ro = pl.ds(off + i * CH, CH)
            yc = (local_buf[ro, :] + recv_ref[r, :].astype(jnp.float32))
            inv = lax.rsqrt(jnp.mean(yc * yc, axis=-1, keepdims=True) + 1e-6)
            out_vmem[ro, :] = (yc * inv * g_vmem[...]).astype(jnp.bfloat16)
            out_cp(i, off, sem_i).start()

        def direct_step(i):
            y_rdma(i).wait_recv()
            x_rdma(i).start()
            fold(recv_direct, i, off_d, i)

        for i in range(NB):
            direct_step(i)

        @pl.when(is_a)
        def _():
            for i in range(NB, NA):
                direct_step(i)

        def fwd_step(i):
            x_rdma(i).wait_recv()
            fold(recv_fwd, i, off_f, NA + i)

        for i in range(NB):
            fwd_step(i)

        @pl.when(my_x == 1)
        def _():
            for i in range(NB, NA):
                fwd_step(i)

        for i in range(NB):
            y_rdma(i).wait_send()
            x_rdma(i).wait_send()
            out_cp(i, off_d, i).wait()
            out_cp(i, off_f, NA + i).wait()

        @pl.when(is_a)
        def _():
            for i in range(NB, NA):
                y_rdma(i).wait_send()
                x_rdma(i).wait_send()
                out_cp(i, off_d, i).wait()

        @pl.when(my_x == 1)
        def _():
            for i in range(NB, NA):
                out_cp(i, off_f, NA + i).wait()

    return pl.pallas_call(
        body,
        out_shape=jax.ShapeDtypeStruct((K, D), jnp.bfloat16),
        in_specs=[pl.BlockSpec(memory_space=pltpu.MemorySpace.HBM),
                  pl.BlockSpec(memory_space=pltpu.MemorySpace.HBM)],
        out_specs=pl.BlockSpec(memory_space=pltpu.MemorySpace.HBM),
        scratch_shapes=[
            pltpu.VMEM((SPLIT, D), jnp.float32),
            pltpu.VMEM((SPLIT, D), jnp.bfloat16),
            pltpu.VMEM((SPLIT, D), jnp.bfloat16),
            pltpu.VMEM((SPLIT, D), jnp.bfloat16),
            pltpu.VMEM((K, D), jnp.float32),
            pltpu.VMEM((K, D), jnp.bfloat16),
            pltpu.VMEM((1, D), jnp.float32),
            pltpu.SemaphoreType.DMA((NA,)),
            pltpu.SemaphoreType.DMA((2,)),
            pltpu.SemaphoreType.DMA((2 * NA,)),
            pltpu.SemaphoreType.DMA,
            pltpu.SemaphoreType.DMA((NA,)),
            pltpu.SemaphoreType.DMA((NA,)),
            pltpu.SemaphoreType.DMA((NA,)),
            pltpu.SemaphoreType.DMA((NA,)),
        ],
        compiler_params=pltpu.CompilerParams(collective_id=0),
    )(partial, g)
